# baseline (speedup 1.0000x reference)
import functools
import math

import jax
import jax.numpy as jnp
from jax.experimental import pallas as pl
from jax.experimental.pallas import tpu as pltpu

_INV_SQRT2 = 1.0 / math.sqrt(2.0)


def _pool_body(x_ref, o_ref):
    s = jnp.sum(x_ref[...], axis=2)

    @pl.when(pl.program_id(0) == 0)
    def _init():
        o_ref[...] = s

    @pl.when(pl.program_id(0) != 0)
    def _acc():
        o_ref[...] += s


def _mlp_body(p_ref, w1_ref, b1_ref, w2_ref, b2_ref, o_ref, *, inv_hw):
    pooled = p_ref[...] * inv_hw
    h = jnp.dot(pooled, w1_ref[...], preferred_element_type=jnp.float32)
    h = h + b1_ref[...]
    h = 0.5 * h * (1.0 + jax.lax.erf(h * _INV_SQRT2))
    logits = jnp.dot(h, w2_ref[...], preferred_element_type=jnp.float32)
    logits = logits + b2_ref[...]
    m = jnp.max(logits, axis=-1, keepdims=True)
    e = jnp.exp(logits - m)
    o_ref[...] = e / jnp.sum(e, axis=-1, keepdims=True)


@jax.jit
def kernel(img_emb, W1, b1, W2, b2):
    B, C, H, W = img_emb.shape
    HW = H * W
    x = img_emb.reshape(B, C, HW)

    HBLK = 256
    grid = (HW // HBLK,)
    pooled = pl.pallas_call(
        _pool_body,
        grid=grid,
        in_specs=[pl.BlockSpec((B, C, HBLK), lambda i: (0, 0, i))],
        out_specs=pl.BlockSpec((B, C), lambda i: (0, 0)),
        out_shape=jax.ShapeDtypeStruct((B, C), jnp.float32),
    )(x)

    out = pl.pallas_call(
        functools.partial(_mlp_body, inv_hw=1.0 / HW),
        in_specs=[
            pl.BlockSpec((B, C), lambda: (0, 0)),
            pl.BlockSpec((C, W1.shape[1]), lambda: (0, 0)),
            pl.BlockSpec((1, W1.shape[1]), lambda: (0, 0)),
            pl.BlockSpec((W1.shape[1], W2.shape[1]), lambda: (0, 0)),
            pl.BlockSpec((1, W2.shape[1]), lambda: (0, 0)),
        ],
        out_specs=pl.BlockSpec((B, W2.shape[1]), lambda: (0, 0)),
        out_shape=jax.ShapeDtypeStruct((B, W2.shape[1]), jnp.float32),
    )(pooled, W1, b1.reshape(1, -1), W2, b2.reshape(1, -1))
    return out
